# Initial kernel scaffold; baseline (speedup 1.0000x reference)
import jax
import jax.numpy as jnp
from jax.experimental import pallas as pl


def kernel(keys_in, values_in, query, Wq, bq, Wk, bk, Wv, bv, Wo, bo, g_q, b_qn, g_o, b_on):
    raise NotImplementedError("write your pallas kernel here")


# speedup vs baseline: 14.5805x; 14.5805x over previous
import math

import jax
import jax.numpy as jnp
from jax.experimental import pallas as pl

B = 64
T = 1024
D = 256
BT = 8


def _ln(x, g, b, eps=1e-5):
    mu = jnp.mean(x, axis=-1, keepdims=True)
    var = jnp.mean((x - mu) * (x - mu), axis=-1, keepdims=True)
    return (x - mu) * jax.lax.rsqrt(var + eps) * g + b


def _body(keys_ref, values_ref, q_ref, Wq_ref, Wk_ref, Wv_ref, Wo_ref,
          bq_ref, bv_ref, bo_ref, gq_ref, bqn_ref, go_ref, bon_ref,
          out_ref):
    qn = _ln(q_ref[:], gq_ref[:], bqn_ref[:])
    qp = jax.lax.dot_general(qn, Wq_ref[:], (((1,), (1,)), ((), ())),
                             preferred_element_type=jnp.float32) + bq_ref[:]
    qk = jax.lax.dot_general(qp, Wk_ref[:], (((1,), (0,)), ((), ())),
                             preferred_element_type=jnp.float32)

    keys = keys_ref[:]
    scores = jax.lax.dot_general(keys, qk, (((2,), (1,)), ((0,), (0,))),
                                 preferred_element_type=jnp.float32)
    scores = scores * (1.0 / math.sqrt(D))

    m = jnp.max(scores, axis=1, keepdims=True)
    e = jnp.exp(scores - m)
    w = e / jnp.sum(e, axis=1, keepdims=True)

    rv = jax.lax.dot_general(w, values_ref[:], (((1,), (1,)), ((0,), (0,))),
                             preferred_element_type=jnp.float32)
    ret = jax.lax.dot_general(rv, Wv_ref[:], (((1,), (1,)), ((), ())),
                              preferred_element_type=jnp.float32) + bv_ref[:]
    out = jax.lax.dot_general(ret, Wo_ref[:], (((1,), (1,)), ((), ())),
                              preferred_element_type=jnp.float32) + bo_ref[:]
    out_ref[:] = _ln(out, go_ref[:], bon_ref[:])


def kernel(keys_in, values_in, query, Wq, bq, Wk, bk, Wv, bv, Wo, bo,
           g_q, b_qn, g_o, b_on):
    del bk
    vecs = [v.reshape(1, D) for v in (bq, bv, bo, g_q, b_qn, g_o, b_on)]
    grid = (B // BT,)
    full = pl.BlockSpec((1, D), lambda i: (0, 0))
    mat = pl.BlockSpec((D, D), lambda i: (0, 0))
    out = pl.pallas_call(
        _body,
        grid=grid,
        in_specs=[
            pl.BlockSpec((BT, T, D), lambda i: (i, 0, 0)),
            pl.BlockSpec((BT, T, D), lambda i: (i, 0, 0)),
            pl.BlockSpec((BT, D), lambda i: (i, 0)),
            mat, mat, mat, mat,
            full, full, full, full, full, full, full,
        ],
        out_specs=pl.BlockSpec((BT, D), lambda i: (i, 0)),
        out_shape=jax.ShapeDtypeStruct((B, D), jnp.float32),
    )(keys_in, values_in, query, Wq, Wk, Wv, Wo, *vecs)
    return out


# speedup vs baseline: 15.9159x; 1.0916x over previous
import math

import jax
import jax.numpy as jnp
from jax.experimental import pallas as pl

B = 64
T = 1024
D = 256
BT = 8


def _ln(x, g, b, eps=1e-5):
    mu = jnp.mean(x, axis=-1, keepdims=True)
    var = jnp.mean((x - mu) * (x - mu), axis=-1, keepdims=True)
    return (x - mu) * jax.lax.rsqrt(var + eps) * g + b


def _body(keys_ref, values_ref, q_ref, Wq_ref, Wk_ref, Wv_ref, Wo_ref,
          bq_ref, bv_ref, bo_ref, gq_ref, bqn_ref, go_ref, bon_ref,
          out_ref):
    qn = _ln(q_ref[:], gq_ref[:], bqn_ref[:])
    qp = jax.lax.dot_general(qn, Wq_ref[:], (((1,), (1,)), ((), ())),
                             preferred_element_type=jnp.float32) + bq_ref[:]
    qk = jax.lax.dot_general(qp, Wk_ref[:], (((1,), (0,)), ((), ())),
                             preferred_element_type=jnp.float32)
    qk = qk * (1.0 / math.sqrt(D))

    keys = keys_ref[:]
    scores = jax.lax.dot_general(keys, qk, (((2,), (1,)), ((0,), (0,))),
                                 preferred_element_type=jnp.float32)

    m = jnp.max(scores, axis=1, keepdims=True)
    e = jnp.exp(scores - m)
    s = jnp.sum(e, axis=1, keepdims=True)

    rv = jax.lax.dot_general(e, values_ref[:], (((1,), (1,)), ((0,), (0,))),
                             preferred_element_type=jnp.float32)
    rv = rv * (1.0 / s)
    ret = jax.lax.dot_general(rv, Wv_ref[:], (((1,), (1,)), ((), ())),
                              preferred_element_type=jnp.float32) + bv_ref[:]
    out = jax.lax.dot_general(ret, Wo_ref[:], (((1,), (1,)), ((), ())),
                              preferred_element_type=jnp.float32) + bo_ref[:]
    out_ref[:] = _ln(out, go_ref[:], bon_ref[:])


def kernel(keys_in, values_in, query, Wq, bq, Wk, bk, Wv, bv, Wo, bo,
           g_q, b_qn, g_o, b_on):
    del bk
    vecs = [v.reshape(1, D) for v in (bq, bv, bo, g_q, b_qn, g_o, b_on)]
    grid = (B // BT,)
    full = pl.BlockSpec((1, D), lambda i: (0, 0))
    mat = pl.BlockSpec((D, D), lambda i: (0, 0))
    out = pl.pallas_call(
        _body,
        grid=grid,
        in_specs=[
            pl.BlockSpec((BT, T, D), lambda i: (i, 0, 0)),
            pl.BlockSpec((BT, T, D), lambda i: (i, 0, 0)),
            pl.BlockSpec((BT, D), lambda i: (i, 0)),
            mat, mat, mat, mat,
            full, full, full, full, full, full, full,
        ],
        out_specs=pl.BlockSpec((BT, D), lambda i: (i, 0)),
        out_shape=jax.ShapeDtypeStruct((B, D), jnp.float32),
    )(keys_in, values_in, query, Wq, Wk, Wv, Wo, *vecs)
    return out


# speedup vs baseline: 17.6985x; 1.1120x over previous
import math

import jax
import jax.numpy as jnp
from jax.experimental import pallas as pl

B = 64
T = 1024
D = 256
BT = 8


def _ln(x, g, b, eps=1e-5):
    mu = jnp.mean(x, axis=-1, keepdims=True)
    var = jnp.mean((x - mu) * (x - mu), axis=-1, keepdims=True)
    return (x - mu) * jax.lax.rsqrt(var + eps) * g + b


def _body(keys_ref, values_ref, q_ref, Wq_ref, Wk_ref, Wv_ref, Wo_ref,
          bq_ref, bv_ref, bo_ref, gq_ref, bqn_ref, go_ref, bon_ref,
          out_ref):
    qn = _ln(q_ref[:], gq_ref[:], bqn_ref[:])
    qp = jax.lax.dot_general(qn, Wq_ref[:], (((1,), (1,)), ((), ())),
                             preferred_element_type=jnp.float32) + bq_ref[:]
    qk = jax.lax.dot_general(qp, Wk_ref[:], (((1,), (0,)), ((), ())),
                             preferred_element_type=jnp.float32)
    qk = qk * (1.0 / math.sqrt(D))

    keys = keys_ref[:]
    scores = jax.lax.dot_general(qk, keys, (((1,), (2,)), ((0,), (0,))),
                                 preferred_element_type=jnp.float32)

    m = jnp.max(scores, axis=1, keepdims=True)
    e = jnp.exp(scores - m)
    s = jnp.sum(e, axis=1, keepdims=True)

    rv = jax.lax.dot_general(e, values_ref[:], (((1,), (1,)), ((0,), (0,))),
                             preferred_element_type=jnp.float32)
    rv = rv * (1.0 / s)
    ret = jax.lax.dot_general(rv, Wv_ref[:], (((1,), (1,)), ((), ())),
                              preferred_element_type=jnp.float32) + bv_ref[:]
    out = jax.lax.dot_general(ret, Wo_ref[:], (((1,), (1,)), ((), ())),
                              preferred_element_type=jnp.float32) + bo_ref[:]
    out_ref[:] = _ln(out, go_ref[:], bon_ref[:])


def kernel(keys_in, values_in, query, Wq, bq, Wk, bk, Wv, bv, Wo, bo,
           g_q, b_qn, g_o, b_on):
    del bk
    vecs = [v.reshape(1, D) for v in (bq, bv, bo, g_q, b_qn, g_o, b_on)]
    grid = (B // BT,)
    full = pl.BlockSpec((1, D), lambda i: (0, 0))
    mat = pl.BlockSpec((D, D), lambda i: (0, 0))
    out = pl.pallas_call(
        _body,
        grid=grid,
        in_specs=[
            pl.BlockSpec((BT, T, D), lambda i: (i, 0, 0)),
            pl.BlockSpec((BT, T, D), lambda i: (i, 0, 0)),
            pl.BlockSpec((BT, D), lambda i: (i, 0)),
            mat, mat, mat, mat,
            full, full, full, full, full, full, full,
        ],
        out_specs=pl.BlockSpec((BT, D), lambda i: (i, 0)),
        out_shape=jax.ShapeDtypeStruct((B, D), jnp.float32),
    )(keys_in, values_in, query, Wq, Wk, Wv, Wo, *vecs)
    return out


# speedup vs baseline: 18.1750x; 1.0269x over previous
import math

import jax
import jax.numpy as jnp
from jax.experimental import pallas as pl

B = 64
T = 1024
D = 256
BT = 8


def _ln(x, g, b, eps=1e-5):
    mu = jnp.mean(x, axis=-1, keepdims=True)
    var = jnp.mean((x - mu) * (x - mu), axis=-1, keepdims=True)
    return (x - mu) * jax.lax.rsqrt(var + eps) * g + b


def _body(keys_ref, keys2_ref, values_ref, values2_ref, q_ref,
          Wq_ref, Wk_ref, Wv_ref, Wo_ref,
          bq_ref, bv_ref, bo_ref, gq_ref, bqn_ref, go_ref, bon_ref,
          out_ref):
    qn = _ln(q_ref[:], gq_ref[:], bqn_ref[:])
    qp = jax.lax.dot_general(qn, Wq_ref[:], (((1,), (1,)), ((), ())),
                             preferred_element_type=jnp.float32) + bq_ref[:]
    qk = jax.lax.dot_general(qp, Wk_ref[:], (((1,), (0,)), ((), ())),
                             preferred_element_type=jnp.float32)
    qk = qk * (1.0 / math.sqrt(D))

    dn = (((1,), (2,)), ((0,), (0,)))
    s1 = jax.lax.dot_general(qk, keys_ref[:], dn,
                             preferred_element_type=jnp.float32)
    s2 = jax.lax.dot_general(qk, keys2_ref[:], dn,
                             preferred_element_type=jnp.float32)

    m = jnp.maximum(jnp.max(s1, axis=1, keepdims=True),
                    jnp.max(s2, axis=1, keepdims=True))
    e1 = jnp.exp(s1 - m)
    e2 = jnp.exp(s2 - m)
    s = (jnp.sum(e1, axis=1, keepdims=True)
         + jnp.sum(e2, axis=1, keepdims=True))

    dn2 = (((1,), (1,)), ((0,), (0,)))
    rv = (jax.lax.dot_general(e1, values_ref[:], dn2,
                              preferred_element_type=jnp.float32)
          + jax.lax.dot_general(e2, values2_ref[:], dn2,
                                preferred_element_type=jnp.float32))
    rv = rv * (1.0 / s)
    ret = jax.lax.dot_general(rv, Wv_ref[:], (((1,), (1,)), ((), ())),
                              preferred_element_type=jnp.float32) + bv_ref[:]
    out = jax.lax.dot_general(ret, Wo_ref[:], (((1,), (1,)), ((), ())),
                              preferred_element_type=jnp.float32) + bo_ref[:]
    out_ref[:] = _ln(out, go_ref[:], bon_ref[:])


def kernel(keys_in, values_in, query, Wq, bq, Wk, bk, Wv, bv, Wo, bo,
           g_q, b_qn, g_o, b_on):
    del bk
    vecs = [v.reshape(1, D) for v in (bq, bv, bo, g_q, b_qn, g_o, b_on)]
    grid = (B // BT,)
    full = pl.BlockSpec((1, D), lambda i: (0, 0))
    mat = pl.BlockSpec((D, D), lambda i: (0, 0))
    out = pl.pallas_call(
        _body,
        grid=grid,
        in_specs=[
            pl.BlockSpec((BT, T // 2, D), lambda i: (i, 0, 0)),
            pl.BlockSpec((BT, T // 2, D), lambda i: (i, 1, 0)),
            pl.BlockSpec((BT, T // 2, D), lambda i: (i, 0, 0)),
            pl.BlockSpec((BT, T // 2, D), lambda i: (i, 1, 0)),
            pl.BlockSpec((BT, D), lambda i: (i, 0)),
            mat, mat, mat, mat,
            full, full, full, full, full, full, full,
        ],
        out_specs=pl.BlockSpec((BT, D), lambda i: (i, 0)),
        out_shape=jax.ShapeDtypeStruct((B, D), jnp.float32),
    )(keys_in, keys_in, values_in, values_in, query, Wq, Wk, Wv, Wo, *vecs)
    return out


# speedup vs baseline: 18.2385x; 1.0035x over previous
import math

import jax
import jax.numpy as jnp
from jax.experimental import pallas as pl

B = 64
T = 1024
D = 256
BT = 8
NSPLIT = 4
TS = T // NSPLIT


def _ln(x, g, b, eps=1e-5):
    mu = jnp.mean(x, axis=-1, keepdims=True)
    var = jnp.mean((x - mu) * (x - mu), axis=-1, keepdims=True)
    return (x - mu) * jax.lax.rsqrt(var + eps) * g + b


def _body(*refs):
    kv_refs = refs[:2 * NSPLIT]
    (q_ref, Wq_ref, Wk_ref, Wv_ref, Wo_ref,
     bq_ref, bv_ref, bo_ref, gq_ref, bqn_ref, go_ref, bon_ref,
     out_ref) = refs[2 * NSPLIT:]
    k_refs = kv_refs[:NSPLIT]
    v_refs = kv_refs[NSPLIT:]

    qn = _ln(q_ref[:], gq_ref[:], bqn_ref[:])
    qp = jax.lax.dot_general(qn, Wq_ref[:], (((1,), (1,)), ((), ())),
                             preferred_element_type=jnp.float32) + bq_ref[:]
    qk = jax.lax.dot_general(qp, Wk_ref[:], (((1,), (0,)), ((), ())),
                             preferred_element_type=jnp.float32)
    qk = qk * (1.0 / math.sqrt(D))

    dn = (((1,), (2,)), ((0,), (0,)))
    ss = [jax.lax.dot_general(qk, kr[:], dn,
                              preferred_element_type=jnp.float32)
          for kr in k_refs]
    m = ss[0].max(axis=1, keepdims=True)
    for si in ss[1:]:
        m = jnp.maximum(m, si.max(axis=1, keepdims=True))
    es = [jnp.exp(si - m) for si in ss]
    s = es[0].sum(axis=1, keepdims=True)
    for ei in es[1:]:
        s = s + ei.sum(axis=1, keepdims=True)

    dn2 = (((1,), (1,)), ((0,), (0,)))
    rv = jax.lax.dot_general(es[0], v_refs[0][:], dn2,
                             preferred_element_type=jnp.float32)
    for ei, vr in zip(es[1:], v_refs[1:]):
        rv = rv + jax.lax.dot_general(ei, vr[:], dn2,
                                      preferred_element_type=jnp.float32)
    rv = rv * (1.0 / s)

    ret = jax.lax.dot_general(rv, Wv_ref[:], (((1,), (1,)), ((), ())),
                              preferred_element_type=jnp.float32) + bv_ref[:]
    out = jax.lax.dot_general(ret, Wo_ref[:], (((1,), (1,)), ((), ())),
                              preferred_element_type=jnp.float32) + bo_ref[:]
    out_ref[:] = _ln(out, go_ref[:], bon_ref[:])


def kernel(keys_in, values_in, query, Wq, bq, Wk, bk, Wv, bv, Wo, bo,
           g_q, b_qn, g_o, b_on):
    del bk
    vecs = [v.reshape(1, D) for v in (bq, bv, bo, g_q, b_qn, g_o, b_on)]
    full = pl.BlockSpec((1, D), lambda i: (0, 0))
    mat = pl.BlockSpec((D, D), lambda i: (0, 0))

    def win(j):
        return pl.BlockSpec((BT, TS, D), lambda i, j=j: (i, j, 0))

    kv_specs = [win(j) for j in range(NSPLIT)] * 2
    out = pl.pallas_call(
        _body,
        grid=(B // BT,),
        in_specs=kv_specs + [
            pl.BlockSpec((BT, D), lambda i: (i, 0)),
            mat, mat, mat, mat,
            full, full, full, full, full, full, full,
        ],
        out_specs=pl.BlockSpec((BT, D), lambda i: (i, 0)),
        out_shape=jax.ShapeDtypeStruct((B, D), jnp.float32),
    )(*([keys_in] * NSPLIT), *([values_in] * NSPLIT), query,
      Wq, Wk, Wv, Wo, *vecs)
    return out
